# Initial kernel scaffold; baseline (speedup 1.0000x reference)
import jax
import jax.numpy as jnp
from jax.experimental import pallas as pl


def kernel(x, pos_table):
    raise NotImplementedError("write your pallas kernel here")


# speedup vs baseline: 1.6677x; 1.6677x over previous
import jax
import jax.numpy as jnp
from jax.experimental import pallas as pl

S_BLK = 1024


def _add_kernel(x_ref, pos_ref, o_ref):
    o_ref[...] = x_ref[...] + pos_ref[...]


def kernel(x, pos_table):
    batch, seq_len, d_model = x.shape
    n_s = seq_len // S_BLK
    return pl.pallas_call(
        _add_kernel,
        grid=(n_s, batch),
        in_specs=[
            pl.BlockSpec((1, S_BLK, d_model), lambda s, b: (b, s, 0)),
            pl.BlockSpec((S_BLK, d_model), lambda s, b: (s, 0)),
        ],
        out_specs=pl.BlockSpec((1, S_BLK, d_model), lambda s, b: (b, s, 0)),
        out_shape=jax.ShapeDtypeStruct((batch, seq_len, d_model), x.dtype),
    )(x, pos_table)


# speedup vs baseline: 1.7344x; 1.0400x over previous
import jax
import jax.numpy as jnp
from jax.experimental import pallas as pl

S_BLK = 2048


def _add_kernel(x_ref, pos_ref, o_ref):
    o_ref[...] = x_ref[...] + pos_ref[...]


def kernel(x, pos_table):
    batch, seq_len, d_model = x.shape
    n_s = seq_len // S_BLK
    return pl.pallas_call(
        _add_kernel,
        grid=(n_s, batch),
        in_specs=[
            pl.BlockSpec((1, S_BLK, d_model), lambda s, b: (b, s, 0)),
            pl.BlockSpec((S_BLK, d_model), lambda s, b: (s, 0)),
        ],
        out_specs=pl.BlockSpec((1, S_BLK, d_model), lambda s, b: (b, s, 0)),
        out_shape=jax.ShapeDtypeStruct((batch, seq_len, d_model), x.dtype),
    )(x, pos_table)
